# baseline (speedup 1.0000x reference)
import functools

import jax
import jax.numpy as jnp
from jax import lax
from jax.experimental import pallas as pl
from jax.experimental.pallas import tpu as pltpu
from jax.experimental.pallas import tpu_sc as plsc

B, P, E = 64, 1024, 768
NC, NS, L = 2, 16, 16
NW = NC * NS
ROWS = P // NW
LANES_PER_ROW = E // L


def _sc_add(patches_hbm, table_hbm, out_hbm, tab_v, buf_v, sem):
    wid = lax.axis_index("s") * NC + lax.axis_index("c")
    p0 = wid * ROWS
    pltpu.sync_copy(table_hbm.at[pl.ds(p0, ROWS)], tab_v)

    def batch_body(b, _):
        pltpu.sync_copy(patches_hbm.at[b, pl.ds(p0, ROWS)], buf_v)

        def row_body(r, _):
            for j in range(LANES_PER_ROW):
                sl = pl.ds(j * L, L)
                buf_v[r, sl] = buf_v[r, sl] + tab_v[r, sl]
            return 0

        lax.fori_loop(0, ROWS, row_body, 0)
        pltpu.sync_copy(buf_v, out_hbm.at[b, pl.ds(p0, ROWS)])
        return 0

    lax.fori_loop(0, B, batch_body, 0)


_sc_kernel = functools.partial(
    pl.kernel,
    out_type=jax.ShapeDtypeStruct((B, P, E), jnp.float32),
    mesh=plsc.VectorSubcoreMesh(core_axis_name="c", subcore_axis_name="s"),
    scratch_types=[
        pltpu.VMEM((ROWS, E), jnp.float32),
        pltpu.VMEM((ROWS, E), jnp.float32),
        pltpu.SemaphoreType.DMA,
    ],
)(_sc_add)


def kernel(patches, table):
    return _sc_kernel(patches, table)


# speedup vs baseline: 1.9468x; 1.9468x over previous
import functools

import jax
import jax.numpy as jnp
from jax import lax
from jax.experimental import pallas as pl
from jax.experimental.pallas import tpu as pltpu
from jax.experimental.pallas import tpu_sc as plsc

B, P, E = 64, 1024, 768
NC, NS, L = 2, 16, 16
NW = NC * NS
ROWS = P // NW
LANES_PER_ROW = E // L


def _sc_add(patches_hbm, table_hbm, out_hbm, tab_v, ibufs, obufs, isems, osems):
    wid = lax.axis_index("s") * NC + lax.axis_index("c")
    p0 = wid * ROWS
    rows = pl.ds(p0, ROWS)
    pltpu.sync_copy(table_hbm.at[rows], tab_v)

    pltpu.make_async_copy(patches_hbm.at[0, rows], ibufs[0], isems[0]).start()
    pltpu.make_async_copy(patches_hbm.at[1, rows], ibufs[1], isems[1]).start()

    def pair_body(i, _):
        for q in range(2):
            ib, ob, si, so = ibufs[q], obufs[q], isems[q], osems[q]
            b = 2 * i + q
            pltpu.make_async_copy(patches_hbm.at[b, rows], ib, si).wait()

            @pl.when(i > 0)
            def _drain():
                pltpu.make_async_copy(ob, out_hbm.at[b, rows], so).wait()

            @plsc.parallel_loop(0, ROWS)
            def row_body(r):
                for j in range(LANES_PER_ROW):
                    sl = pl.ds(j * L, L)
                    ob[r, sl] = ib[r, sl] + tab_v[r, sl]

            pltpu.make_async_copy(ob, out_hbm.at[b, rows], so).start()

            @pl.when(b + 2 < B)
            def _prefetch():
                pltpu.make_async_copy(patches_hbm.at[b + 2, rows], ib, si).start()

        return 0

    lax.fori_loop(0, B // 2, pair_body, 0)
    pltpu.make_async_copy(obufs[0], out_hbm.at[B - 2, rows], osems[0]).wait()
    pltpu.make_async_copy(obufs[1], out_hbm.at[B - 1, rows], osems[1]).wait()


def _sc_body(patches_hbm, table_hbm, out_hbm, tab_v,
             ibuf0, ibuf1, obuf0, obuf1, isem0, isem1, osem0, osem1):
    _sc_add(patches_hbm, table_hbm, out_hbm, tab_v,
            (ibuf0, ibuf1), (obuf0, obuf1), (isem0, isem1), (osem0, osem1))


_sc_kernel = functools.partial(
    pl.kernel,
    out_type=jax.ShapeDtypeStruct((B, P, E), jnp.float32),
    mesh=plsc.VectorSubcoreMesh(core_axis_name="c", subcore_axis_name="s"),
    scratch_types=[
        pltpu.VMEM((ROWS, E), jnp.float32),
        pltpu.VMEM((ROWS, E), jnp.float32),
        pltpu.VMEM((ROWS, E), jnp.float32),
        pltpu.VMEM((ROWS, E), jnp.float32),
        pltpu.VMEM((ROWS, E), jnp.float32),
        pltpu.SemaphoreType.DMA,
        pltpu.SemaphoreType.DMA,
        pltpu.SemaphoreType.DMA,
        pltpu.SemaphoreType.DMA,
    ],
)(_sc_body)


def kernel(patches, table):
    return _sc_kernel(patches, table)
